# baseline (speedup 1.0000x reference)
import functools

import jax
import jax.numpy as jnp
from jax import lax
from jax.experimental import pallas as pl
from jax.experimental.pallas import tpu as pltpu
from jax.experimental.pallas import tpu_sc as plsc

L = 16
IDW = 100
CHUNK = 200
NSEG = 10000


def _sc_body(nchunk, nsub, cwidth, chalf,
             xab, src2, tar2, xba, out_p, out_a, out_b,
             xb0, xb1, yb0, yb1, oa0, oa1, ob0, ob1, op0, op1,
             ga0, ga1, gb0, gb1,
             si0, si1, si2, ti0, ti1, ti2,
             in_s0, in_s1, out_s0, out_s1, g_s0, g_s1,
             ix_s0, ix_s1, ix_s2,
             acc_s, acc_t):
    c = lax.axis_index("c")
    s = lax.axis_index("s")
    nk = chalf // L
    xb = (xb0, xb1)
    yb = (yb0, yb1)
    oa = (oa0, oa1)
    ob = (ob0, ob1)
    op = (op0, op1)
    ga = (ga0, ga1)
    gb = (gb0, gb1)
    si = (si0, si1, si2)
    ti = (ti0, ti1, ti2)
    in_s = (in_s0, in_s1)
    out_s = (out_s0, out_s1)
    g_s = (g_s0, g_s1)
    ix_s = (ix_s0, ix_s1, ix_s2)

    for sub in range(cwidth // chalf):
        _sweep(nchunk, nsub, chalf, c * cwidth + sub * chalf, s, nk,
               xab, src2, tar2, xba, out_p, out_a, out_b,
               xb, yb, oa, ob, op, ga, gb, si, ti,
               in_s, out_s, g_s, ix_s, acc_s, acc_t)


def _sweep(nchunk, nsub, chalf, c0, s, nk,
           xab, src2, tar2, xba, out_p, out_a, out_b,
           xb, yb, oa, ob, op, ga, gb, si, ti,
           in_s, out_s, g_s, ix_s, acc_s, acc_t):
    xb0 = xb[0]

    def xslice(ref, base):
        return ref.at[pl.ds(base, CHUNK), pl.ds(c0, chalf)]

    ntrip = nchunk // nsub

    def chunk_idx(t):
        return s * ntrip + t

    def fire_in(t, b, q):
        ci = chunk_idx(t)
        pltpu.async_copy(xslice(xab, ci * CHUNK), xb[b], in_s[b])
        pltpu.async_copy(xslice(xba, ci * CHUNK), yb[b], in_s[b])
        pltpu.async_copy(src2.at[pl.ds(ci * 2, 2), :], si[q], ix_s[q])
        pltpu.async_copy(tar2.at[pl.ds(ci * 2, 2), :], ti[q], ix_s[q])

    def wait_in(b):
        pltpu.make_async_copy(xslice(xab, 0), xb[b], in_s[b]).wait()
        pltpu.make_async_copy(xslice(xba, 0), yb[b], in_s[b]).wait()

    def wait_idx(q):
        pltpu.make_async_copy(src2.at[pl.ds(0, 2), :], si[q], ix_s[q]).wait()
        pltpu.make_async_copy(tar2.at[pl.ds(0, 2), :], ti[q], ix_s[q]).wait()

    def fire_scatter(b, q):
        for j in range(2):
            pltpu.async_copy(oa[b].at[pl.ds(j * IDW, IDW)],
                             acc_s.at[si[q].at[j]], out_s[b], add=True)
            pltpu.async_copy(ob[b].at[pl.ds(j * IDW, IDW)],
                             acc_t.at[ti[q].at[j]], out_s[b], add=True)

    def wait_scatter(b, q):
        for j in range(2):
            pltpu.make_async_copy(oa[b].at[pl.ds(j * IDW, IDW)],
                                  acc_s.at[si[q].at[j]], out_s[b]).wait()
            pltpu.make_async_copy(ob[b].at[pl.ds(j * IDW, IDW)],
                                  acc_t.at[ti[q].at[j]], out_s[b]).wait()

    def fire_gather(b, q):
        for j in range(2):
            pltpu.async_copy(acc_s.at[si[q].at[j]],
                             ga[b].at[pl.ds(j * IDW, IDW)], g_s[b])
            pltpu.async_copy(acc_t.at[ti[q].at[j]],
                             gb[b].at[pl.ds(j * IDW, IDW)], g_s[b])

    def wait_gather(b, q):
        pltpu.make_async_copy(acc_s.at[si[q].at[0]], ga[b], g_s[b]).wait()
        pltpu.make_async_copy(acc_t.at[ti[q].at[0]], gb[b], g_s[b]).wait()

    def fire_out(t, b):
        base = chunk_idx(t) * CHUNK
        pltpu.async_copy(op[b], xslice(out_p, base), out_s[b])
        pltpu.async_copy(oa[b], xslice(out_a, base), out_s[b])
        pltpu.async_copy(ob[b], xslice(out_b, base), out_s[b])

    def wait_out(b):
        pltpu.make_async_copy(op[b], xslice(out_p, 0), out_s[b]).wait()
        pltpu.make_async_copy(oa[b], xslice(out_a, 0), out_s[b]).wait()
        pltpu.make_async_copy(ob[b], xslice(out_b, 0), out_s[b]).wait()

    rows_sub = NSEG // nsub
    zblk = 125

    @plsc.parallel_loop(0, zblk)
    def _(r):
        for k in range(nk):
            xb0[r, pl.ds(k * L, L)] = jnp.zeros((L,), jnp.float32)

    def zcopy(b, _):
        r0 = s * rows_sub + b * zblk
        pltpu.sync_copy(xb0.at[pl.ds(0, zblk)], acc_s.at[pl.ds(r0, zblk)])
        pltpu.sync_copy(xb0.at[pl.ds(0, zblk)], acc_t.at[pl.ds(r0, zblk)])
        return 0

    lax.fori_loop(0, rows_sub // zblk, zcopy, 0)
    plsc.subcore_barrier()

    _scope_a = jax.named_scope("phase_a")
    _scope_a.__enter__()
    fire_in(0, 0, 0)
    fire_in(1, 1, 1)

    def phase_a(tt, _):
        for q in range(6):
            b = q % 2
            q3 = q % 3
            t = tt * 6 + q

            @pl.when(t < ntrip)
            def _():
                wait_in(b)
                wait_idx(q3)

                @plsc.parallel_loop(0, CHUNK, unroll=4)
                def _(r):
                    for k in range(nk):
                        oa[b][r, pl.ds(k * L, L)] = jnp.exp(xb[b][r, pl.ds(k * L, L)])
                        ob[b][r, pl.ds(k * L, L)] = jnp.exp(yb[b][r, pl.ds(k * L, L)])

                @pl.when(t >= 1)
                def _():
                    wait_scatter(1 - b, (q3 + 2) % 3)

                fire_scatter(b, q3)

                @pl.when(t + 2 < ntrip)
                def _():
                    fire_in(t + 2, b, (q3 + 2) % 3)
        return 0

    lax.fori_loop(0, (ntrip + 5) // 6, phase_a, 0)
    wait_scatter((ntrip - 1) % 2, (ntrip - 1) % 3)
    plsc.subcore_barrier()
    _scope_a.__exit__(None, None, None)

    _scope_b = jax.named_scope("phase_b")
    _scope_b.__enter__()
    fire_in(0, 0, 0)
    fire_in(1, 1, 1)
    wait_idx(0)
    fire_gather(0, 0)

    def phase_b(tt, _):
        for q in range(6):
            b = q % 2
            q3 = q % 3
            t = tt * 6 + q

            @pl.when(t < ntrip)
            def _():
                @pl.when(t + 1 < ntrip)
                def _():
                    wait_idx((q3 + 1) % 3)
                    fire_gather(1 - b, (q3 + 1) % 3)

                wait_in(b)
                wait_gather(b, q3)

                @pl.when(t >= 2)
                def _():
                    wait_out(b)

                @plsc.parallel_loop(0, CHUNK, unroll=4)
                def _(r):
                    for k in range(nk):
                        za = jnp.exp(xb[b][r, pl.ds(k * L, L)]) / ga[b][r, pl.ds(k * L, L)]
                        zb_ = jnp.exp(yb[b][r, pl.ds(k * L, L)]) / gb[b][r, pl.ds(k * L, L)]
                        oa[b][r, pl.ds(k * L, L)] = za
                        ob[b][r, pl.ds(k * L, L)] = zb_
                        op[b][r, pl.ds(k * L, L)] = za * zb_

                fire_out(t, b)

                @pl.when(t + 2 < ntrip)
                def _():
                    fire_in(t + 2, b, (q3 + 2) % 3)
        return 0

    lax.fori_loop(0, (ntrip + 5) // 6, phase_b, 0)
    wait_out(0)
    wait_out(1)
    plsc.subcore_barrier()
    _scope_b.__exit__(None, None, None)


def kernel(xab, src_id, tar_id, xba):
    E, C = xab.shape
    info = plsc.get_sparse_core_info()
    nc, ns = info.num_cores, info.num_subcores
    cwidth = C // nc
    chalf = cwidth // 2
    nchunk = E // CHUNK
    src2 = src_id.reshape(E // IDW, IDW)
    tar2 = tar_id.reshape(E // IDW, IDW)
    mesh = plsc.VectorSubcoreMesh(core_axis_name="c", subcore_axis_name="s")
    out_type = (jax.ShapeDtypeStruct((E, C), jnp.float32),) * 3
    buf = pltpu.VMEM((CHUNK, chalf), jnp.float32)
    ibuf = pltpu.VMEM((2, IDW), jnp.int32)
    f = pl.kernel(
        functools.partial(_sc_body, nchunk, ns, cwidth, chalf),
        out_type=out_type,
        mesh=mesh,
        compiler_params=pltpu.CompilerParams(use_tc_tiling_on_sc=False),
        scratch_types=(
            [buf] * 14
            + [ibuf] * 6
            + [pltpu.SemaphoreType.DMA] * 9
            + [
                pltpu.VMEM_SHARED((NSEG, chalf), jnp.float32),
                pltpu.VMEM_SHARED((NSEG, chalf), jnp.float32),
            ]
        ),
    )
    return f(xab, src2, tar2, xba)


# speedup vs baseline: 1.0008x; 1.0008x over previous
import functools

import jax
import jax.numpy as jnp
from jax import lax
from jax.experimental import pallas as pl
from jax.experimental.pallas import tpu as pltpu
from jax.experimental.pallas import tpu_sc as plsc

L = 16
IDW = 100
CHUNK = 200
NSEG = 10000


def _sc_body(nchunk, nsub, cwidth, chalf,
             xab, src2, tar2, xba, out_p, out_a, out_b,
             xb0, xb1, yb0, yb1, oa0, oa1, ob0, ob1, op0, op1,
             ga0, ga1, gb0, gb1,
             si0, si1, si2, ti0, ti1, ti2,
             in_s0, in_s1, out_s0, out_s1, g_s0, g_s1,
             ix_s0, ix_s1, ix_s2,
             acc_s, acc_t):
    c = lax.axis_index("c")
    s = lax.axis_index("s")
    nk = chalf // L
    xb = (xb0, xb1)
    yb = (yb0, yb1)
    oa = (oa0, oa1)
    ob = (ob0, ob1)
    op = (op0, op1)
    ga = (ga0, ga1)
    gb = (gb0, gb1)
    si = (si0, si1, si2)
    ti = (ti0, ti1, ti2)
    in_s = (in_s0, in_s1)
    out_s = (out_s0, out_s1)
    g_s = (g_s0, g_s1)
    ix_s = (ix_s0, ix_s1, ix_s2)

    for sub in range(cwidth // chalf):
        _sweep(nchunk, nsub, chalf, c * cwidth + sub * chalf, s, nk,
               xab, src2, tar2, xba, out_p, out_a, out_b,
               xb, yb, oa, ob, op, ga, gb, si, ti,
               in_s, out_s, g_s, ix_s, acc_s, acc_t)


def _sweep(nchunk, nsub, chalf, c0, s, nk,
           xab, src2, tar2, xba, out_p, out_a, out_b,
           xb, yb, oa, ob, op, ga, gb, si, ti,
           in_s, out_s, g_s, ix_s, acc_s, acc_t):
    xb0 = xb[0]

    def xslice(ref, base):
        return ref.at[pl.ds(base, CHUNK), pl.ds(c0, chalf)]

    ntrip = nchunk // nsub

    def chunk_idx(t):
        return s + t * nsub

    def fire_in(t, b, q):
        ci = chunk_idx(t)
        pltpu.async_copy(xslice(xab, ci * CHUNK), xb[b], in_s[b])
        pltpu.async_copy(xslice(xba, ci * CHUNK), yb[b], in_s[b])
        pltpu.async_copy(src2.at[pl.ds(ci * 2, 2), :], si[q], ix_s[q])
        pltpu.async_copy(tar2.at[pl.ds(ci * 2, 2), :], ti[q], ix_s[q])

    def wait_in(b):
        pltpu.make_async_copy(xslice(xab, 0), xb[b], in_s[b]).wait()
        pltpu.make_async_copy(xslice(xba, 0), yb[b], in_s[b]).wait()

    def wait_idx(q):
        pltpu.make_async_copy(src2.at[pl.ds(0, 2), :], si[q], ix_s[q]).wait()
        pltpu.make_async_copy(tar2.at[pl.ds(0, 2), :], ti[q], ix_s[q]).wait()

    def fire_scatter(b, q):
        for j in range(2):
            pltpu.async_copy(oa[b].at[pl.ds(j * IDW, IDW)],
                             acc_s.at[si[q].at[j]], out_s[b], add=True)
            pltpu.async_copy(ob[b].at[pl.ds(j * IDW, IDW)],
                             acc_t.at[ti[q].at[j]], out_s[b], add=True)

    def wait_scatter(b, q):
        for j in range(2):
            pltpu.make_async_copy(oa[b].at[pl.ds(j * IDW, IDW)],
                                  acc_s.at[si[q].at[j]], out_s[b]).wait()
            pltpu.make_async_copy(ob[b].at[pl.ds(j * IDW, IDW)],
                                  acc_t.at[ti[q].at[j]], out_s[b]).wait()

    def fire_gather(b, q):
        for j in range(2):
            pltpu.async_copy(acc_s.at[si[q].at[j]],
                             ga[b].at[pl.ds(j * IDW, IDW)], g_s[b])
            pltpu.async_copy(acc_t.at[ti[q].at[j]],
                             gb[b].at[pl.ds(j * IDW, IDW)], g_s[b])

    def wait_gather(b, q):
        pltpu.make_async_copy(acc_s.at[si[q].at[0]], ga[b], g_s[b]).wait()
        pltpu.make_async_copy(acc_t.at[ti[q].at[0]], gb[b], g_s[b]).wait()

    def fire_out(t, b):
        base = chunk_idx(t) * CHUNK
        pltpu.async_copy(op[b], xslice(out_p, base), out_s[b])
        pltpu.async_copy(oa[b], xslice(out_a, base), out_s[b])
        pltpu.async_copy(ob[b], xslice(out_b, base), out_s[b])

    def wait_out(b):
        pltpu.make_async_copy(op[b], xslice(out_p, 0), out_s[b]).wait()
        pltpu.make_async_copy(oa[b], xslice(out_a, 0), out_s[b]).wait()
        pltpu.make_async_copy(ob[b], xslice(out_b, 0), out_s[b]).wait()

    rows_sub = NSEG // nsub
    zblk = 125

    @plsc.parallel_loop(0, zblk)
    def _(r):
        for k in range(nk):
            xb0[r, pl.ds(k * L, L)] = jnp.zeros((L,), jnp.float32)

    def zcopy(b, _):
        r0 = s * rows_sub + b * zblk
        pltpu.sync_copy(xb0.at[pl.ds(0, zblk)], acc_s.at[pl.ds(r0, zblk)])
        pltpu.sync_copy(xb0.at[pl.ds(0, zblk)], acc_t.at[pl.ds(r0, zblk)])
        return 0

    lax.fori_loop(0, rows_sub // zblk, zcopy, 0)
    plsc.subcore_barrier()

    _scope_a = jax.named_scope("phase_a")
    _scope_a.__enter__()
    fire_in(0, 0, 0)
    fire_in(1, 1, 1)

    def phase_a(tt, _):
        for q in range(6):
            b = q % 2
            q3 = q % 3
            t = tt * 6 + q

            @pl.when(t < ntrip)
            def _():
                wait_in(b)
                wait_idx(q3)

                @plsc.parallel_loop(0, CHUNK, unroll=4)
                def _(r):
                    for k in range(nk):
                        oa[b][r, pl.ds(k * L, L)] = jnp.exp(xb[b][r, pl.ds(k * L, L)])
                        ob[b][r, pl.ds(k * L, L)] = jnp.exp(yb[b][r, pl.ds(k * L, L)])

                @pl.when(t >= 1)
                def _():
                    wait_scatter(1 - b, (q3 + 2) % 3)

                fire_scatter(b, q3)

                @pl.when(t + 2 < ntrip)
                def _():
                    fire_in(t + 2, b, (q3 + 2) % 3)
        return 0

    lax.fori_loop(0, (ntrip + 5) // 6, phase_a, 0)
    wait_scatter((ntrip - 1) % 2, (ntrip - 1) % 3)
    plsc.subcore_barrier()
    _scope_a.__exit__(None, None, None)

    _scope_b = jax.named_scope("phase_b")
    _scope_b.__enter__()
    fire_in(0, 0, 0)
    fire_in(1, 1, 1)
    wait_idx(0)
    fire_gather(0, 0)

    def phase_b(tt, _):
        for q in range(6):
            b = q % 2
            q3 = q % 3
            t = tt * 6 + q

            @pl.when(t < ntrip)
            def _():
                @pl.when(t + 1 < ntrip)
                def _():
                    wait_idx((q3 + 1) % 3)
                    fire_gather(1 - b, (q3 + 1) % 3)

                wait_in(b)
                wait_gather(b, q3)

                @pl.when(t >= 2)
                def _():
                    wait_out(b)

                @plsc.parallel_loop(0, CHUNK, unroll=4)
                def _(r):
                    for k in range(nk):
                        za = jnp.exp(xb[b][r, pl.ds(k * L, L)]) / ga[b][r, pl.ds(k * L, L)]
                        zb_ = jnp.exp(yb[b][r, pl.ds(k * L, L)]) / gb[b][r, pl.ds(k * L, L)]
                        oa[b][r, pl.ds(k * L, L)] = za
                        ob[b][r, pl.ds(k * L, L)] = zb_
                        op[b][r, pl.ds(k * L, L)] = za * zb_

                fire_out(t, b)

                @pl.when(t + 2 < ntrip)
                def _():
                    fire_in(t + 2, b, (q3 + 2) % 3)
        return 0

    lax.fori_loop(0, (ntrip + 5) // 6, phase_b, 0)
    wait_out(0)
    wait_out(1)
    plsc.subcore_barrier()
    _scope_b.__exit__(None, None, None)


def kernel(xab, src_id, tar_id, xba):
    E, C = xab.shape
    info = plsc.get_sparse_core_info()
    nc, ns = info.num_cores, info.num_subcores
    cwidth = C // nc
    chalf = cwidth // 2
    nchunk = E // CHUNK
    src2 = src_id.reshape(E // IDW, IDW)
    tar2 = tar_id.reshape(E // IDW, IDW)
    mesh = plsc.VectorSubcoreMesh(core_axis_name="c", subcore_axis_name="s")
    out_type = (jax.ShapeDtypeStruct((E, C), jnp.float32),) * 3
    buf = pltpu.VMEM((CHUNK, chalf), jnp.float32)
    ibuf = pltpu.VMEM((2, IDW), jnp.int32)
    f = pl.kernel(
        functools.partial(_sc_body, nchunk, ns, cwidth, chalf),
        out_type=out_type,
        mesh=mesh,
        compiler_params=pltpu.CompilerParams(use_tc_tiling_on_sc=False),
        scratch_types=(
            [buf] * 14
            + [ibuf] * 6
            + [pltpu.SemaphoreType.DMA] * 9
            + [
                pltpu.VMEM_SHARED((NSEG, chalf), jnp.float32),
                pltpu.VMEM_SHARED((NSEG, chalf), jnp.float32),
            ]
        ),
    )
    return f(xab, src2, tar2, xba)


# speedup vs baseline: 1.0172x; 1.0163x over previous
import functools

import jax
import jax.numpy as jnp
from jax import lax
from jax.experimental import pallas as pl
from jax.experimental.pallas import tpu as pltpu
from jax.experimental.pallas import tpu_sc as plsc

L = 16
IDW = 100
CHUNK = 200
NSEG = 10000


def _sc_body(nchunk, nsub, cwidth, chalf,
             xab, src2, tar2, xba, out_p, out_a, out_b,
             xb0, xb1, yb0, yb1, oa0, oa1, ob0, ob1, op0, op1,
             ga0, ga1, gb0, gb1,
             si0, si1, si2, ti0, ti1, ti2,
             in_s0, in_s1, out_s0, out_s1, g_s0, g_s1,
             ix_s0, ix_s1, ix_s2,
             acc_s, acc_t):
    c = lax.axis_index("c")
    s = lax.axis_index("s")
    nk = chalf // L
    xb = (xb0, xb1)
    yb = (yb0, yb1)
    oa = (oa0, oa1)
    ob = (ob0, ob1)
    op = (op0, op1)
    ga = (ga0, ga1)
    gb = (gb0, gb1)
    si = (si0, si1, si2)
    ti = (ti0, ti1, ti2)
    in_s = (in_s0, in_s1)
    out_s = (out_s0, out_s1)
    g_s = (g_s0, g_s1)
    ix_s = (ix_s0, ix_s1, ix_s2)

    for sub in range(cwidth // chalf):
        _sweep(nchunk, nsub, chalf, c * cwidth + sub * chalf, s, nk,
               xab, src2, tar2, xba, out_p, out_a, out_b,
               xb, yb, oa, ob, op, ga, gb, si, ti,
               in_s, out_s, g_s, ix_s, acc_s, acc_t)


def _sweep(nchunk, nsub, chalf, c0, s, nk,
           xab, src2, tar2, xba, out_p, out_a, out_b,
           xb, yb, oa, ob, op, ga, gb, si, ti,
           in_s, out_s, g_s, ix_s, acc_s, acc_t):
    xb0 = xb[0]

    def xslice(ref, base):
        return ref.at[pl.ds(base, CHUNK), pl.ds(c0, chalf)]

    ntrip = nchunk // nsub

    def chunk_idx(t):
        return s + t * nsub

    def fire_in(t, b, q):
        ci = chunk_idx(t)
        pltpu.async_copy(xslice(xab, ci * CHUNK), xb[b], in_s[b])
        pltpu.async_copy(xslice(xba, ci * CHUNK), yb[b], in_s[b])
        pltpu.async_copy(src2.at[pl.ds(ci * 2, 2), :], si[q], ix_s[q])
        pltpu.async_copy(tar2.at[pl.ds(ci * 2, 2), :], ti[q], ix_s[q])

    def wait_in(b):
        pltpu.make_async_copy(xslice(xab, 0), xb[b], in_s[b]).wait()
        pltpu.make_async_copy(xslice(xba, 0), yb[b], in_s[b]).wait()

    def wait_idx(q):
        pltpu.make_async_copy(src2.at[pl.ds(0, 2), :], si[q], ix_s[q]).wait()
        pltpu.make_async_copy(tar2.at[pl.ds(0, 2), :], ti[q], ix_s[q]).wait()

    def fire_scatter(b, q):
        for j in range(2):
            pltpu.async_copy(oa[b].at[pl.ds(j * IDW, IDW)],
                             acc_s.at[si[q].at[j]], out_s[b], add=True)
            pltpu.async_copy(ob[b].at[pl.ds(j * IDW, IDW)],
                             acc_t.at[ti[q].at[j]], out_s[b], add=True)

    def wait_scatter(b, q):
        for j in range(2):
            pltpu.make_async_copy(oa[b].at[pl.ds(j * IDW, IDW)],
                                  acc_s.at[si[q].at[j]], out_s[b]).wait()
            pltpu.make_async_copy(ob[b].at[pl.ds(j * IDW, IDW)],
                                  acc_t.at[ti[q].at[j]], out_s[b]).wait()

    def fire_gather(b, q):
        for j in range(2):
            pltpu.async_copy(acc_s.at[si[q].at[j]],
                             ga[b].at[pl.ds(j * IDW, IDW)], g_s[b])
            pltpu.async_copy(acc_t.at[ti[q].at[j]],
                             gb[b].at[pl.ds(j * IDW, IDW)], g_s[b])

    def wait_gather(b, q):
        pltpu.make_async_copy(acc_s.at[si[q].at[0]], ga[b], g_s[b]).wait()
        pltpu.make_async_copy(acc_t.at[ti[q].at[0]], gb[b], g_s[b]).wait()

    def fire_out(t, b):
        base = chunk_idx(t) * CHUNK
        pltpu.async_copy(op[b], xslice(out_p, base), out_s[b])
        pltpu.async_copy(oa[b], xslice(out_a, base), out_s[b])
        pltpu.async_copy(ob[b], xslice(out_b, base), out_s[b])

    def wait_out(b):
        pltpu.make_async_copy(op[b], xslice(out_p, 0), out_s[b]).wait()
        pltpu.make_async_copy(oa[b], xslice(out_a, 0), out_s[b]).wait()
        pltpu.make_async_copy(ob[b], xslice(out_b, 0), out_s[b]).wait()

    rows_sub = NSEG // nsub
    zblk = 125

    @plsc.parallel_loop(0, zblk)
    def _(r):
        for k in range(nk):
            xb0[r, pl.ds(k * L, L)] = jnp.zeros((L,), jnp.float32)

    def zcopy(b, _):
        r0 = s * rows_sub + b * zblk
        pltpu.sync_copy(xb0.at[pl.ds(0, zblk)], acc_s.at[pl.ds(r0, zblk)])
        pltpu.sync_copy(xb0.at[pl.ds(0, zblk)], acc_t.at[pl.ds(r0, zblk)])
        return 0

    lax.fori_loop(0, rows_sub // zblk, zcopy, 0)
    plsc.subcore_barrier()

    _scope_a = jax.named_scope("phase_a")
    _scope_a.__enter__()
    fire_in(0, 0, 0)
    fire_in(1, 1, 1)

    def phase_a(tt, _):
        for q in range(6):
            b = q % 2
            q3 = q % 3
            t = tt * 6 + q

            @pl.when(t < ntrip)
            def _():
                wait_in(b)
                wait_idx(q3)

                @plsc.parallel_loop(0, CHUNK, unroll=4)
                def _(r):
                    for k in range(nk):
                        oa[b][r, pl.ds(k * L, L)] = jnp.exp(xb[b][r, pl.ds(k * L, L)])
                        ob[b][r, pl.ds(k * L, L)] = jnp.exp(yb[b][r, pl.ds(k * L, L)])

                @pl.when(t >= 1)
                def _():
                    wait_scatter(1 - b, (q3 + 2) % 3)

                fire_scatter(b, q3)

                @pl.when(t + 2 < ntrip)
                def _():
                    fire_in(t + 2, b, (q3 + 2) % 3)
        return 0

    lax.fori_loop(0, (ntrip + 5) // 6, phase_a, 0)
    wait_scatter((ntrip - 1) % 2, (ntrip - 1) % 3)
    plsc.subcore_barrier()
    _scope_a.__exit__(None, None, None)

    ga0, gb0 = ga[0], gb[0]

    def rec_block(blk, _):
        r0 = s * rows_sub + blk * zblk
        pltpu.sync_copy(acc_s.at[pl.ds(r0, zblk)], ga0.at[pl.ds(0, zblk)])
        pltpu.sync_copy(acc_t.at[pl.ds(r0, zblk)], gb0.at[pl.ds(0, zblk)])

        @plsc.parallel_loop(0, zblk)
        def _(r):
            for k in range(nk):
                ga0[r, pl.ds(k * L, L)] = 1.0 / ga0[r, pl.ds(k * L, L)]
                gb0[r, pl.ds(k * L, L)] = 1.0 / gb0[r, pl.ds(k * L, L)]

        pltpu.sync_copy(ga0.at[pl.ds(0, zblk)], acc_s.at[pl.ds(r0, zblk)])
        pltpu.sync_copy(gb0.at[pl.ds(0, zblk)], acc_t.at[pl.ds(r0, zblk)])
        return 0

    lax.fori_loop(0, rows_sub // zblk, rec_block, 0)
    plsc.subcore_barrier()

    _scope_b = jax.named_scope("phase_b")
    _scope_b.__enter__()
    fire_in(0, 0, 0)
    fire_in(1, 1, 1)
    wait_idx(0)
    fire_gather(0, 0)

    def phase_b(tt, _):
        for q in range(6):
            b = q % 2
            q3 = q % 3
            t = tt * 6 + q

            @pl.when(t < ntrip)
            def _():
                @pl.when(t + 1 < ntrip)
                def _():
                    wait_idx((q3 + 1) % 3)
                    fire_gather(1 - b, (q3 + 1) % 3)

                wait_in(b)
                wait_gather(b, q3)

                @pl.when(t >= 2)
                def _():
                    wait_out(b)

                @plsc.parallel_loop(0, CHUNK, unroll=4)
                def _(r):
                    for k in range(nk):
                        za = jnp.exp(xb[b][r, pl.ds(k * L, L)]) * ga[b][r, pl.ds(k * L, L)]
                        zb_ = jnp.exp(yb[b][r, pl.ds(k * L, L)]) * gb[b][r, pl.ds(k * L, L)]
                        oa[b][r, pl.ds(k * L, L)] = za
                        ob[b][r, pl.ds(k * L, L)] = zb_
                        op[b][r, pl.ds(k * L, L)] = za * zb_

                fire_out(t, b)

                @pl.when(t + 2 < ntrip)
                def _():
                    fire_in(t + 2, b, (q3 + 2) % 3)
        return 0

    lax.fori_loop(0, (ntrip + 5) // 6, phase_b, 0)
    wait_out(0)
    wait_out(1)
    plsc.subcore_barrier()
    _scope_b.__exit__(None, None, None)


def kernel(xab, src_id, tar_id, xba):
    E, C = xab.shape
    info = plsc.get_sparse_core_info()
    nc, ns = info.num_cores, info.num_subcores
    cwidth = C // nc
    chalf = cwidth // 2
    nchunk = E // CHUNK
    src2 = src_id.reshape(E // IDW, IDW)
    tar2 = tar_id.reshape(E // IDW, IDW)
    mesh = plsc.VectorSubcoreMesh(core_axis_name="c", subcore_axis_name="s")
    out_type = (jax.ShapeDtypeStruct((E, C), jnp.float32),) * 3
    buf = pltpu.VMEM((CHUNK, chalf), jnp.float32)
    ibuf = pltpu.VMEM((2, IDW), jnp.int32)
    f = pl.kernel(
        functools.partial(_sc_body, nchunk, ns, cwidth, chalf),
        out_type=out_type,
        mesh=mesh,
        compiler_params=pltpu.CompilerParams(use_tc_tiling_on_sc=False),
        scratch_types=(
            [buf] * 14
            + [ibuf] * 6
            + [pltpu.SemaphoreType.DMA] * 9
            + [
                pltpu.VMEM_SHARED((NSEG, chalf), jnp.float32),
                pltpu.VMEM_SHARED((NSEG, chalf), jnp.float32),
            ]
        ),
    )
    return f(xab, src2, tar2, xba)
